# Initial kernel scaffold; baseline (speedup 1.0000x reference)
import jax
import jax.numpy as jnp
from jax.experimental import pallas as pl


def kernel(h, cand_edge_feat, u_1, c_adj, j, W_phi, b_phi, W1, b1, W2, b2, W3, b3, Wv, bv):
    raise NotImplementedError("write your pallas kernel here")


# speedup vs baseline: 2.1979x; 2.1979x over previous
import functools

import jax
import jax.numpy as jnp
from jax import lax
from jax.experimental import pallas as pl
from jax.experimental.pallas import tpu as pltpu
from jax.experimental.pallas import tpu_sc as plsc

F32 = jnp.float32

N = 10000
E = 160000
M = 160000
FD = 128


EB = 2000


def _embed_body(feat_ref, wt_ref, b_ref, out_ref):
    out_ref[...] = (
        jnp.dot(feat_ref[...], wt_ref[...], preferred_element_type=F32)
        + b_ref[...]
    )


def _edge_embed(feat, w_phi_t_pad, b_phi_row_pad):
    return pl.pallas_call(
        _embed_body,
        grid=(E // EB,),
        in_specs=[
            pl.BlockSpec((EB, FD), lambda i: (i, 0)),
            pl.BlockSpec((FD, 128), lambda i: (0, 0)),
            pl.BlockSpec((1, 128), lambda i: (0, 0)),
        ],
        out_specs=pl.BlockSpec((EB, 128), lambda i: (i, 0)),
        out_shape=jax.ShapeDtypeStruct((E, 128), F32),
    )(feat, w_phi_t_pad, b_phi_row_pad)


NC, NS = 2, 16
NW = NC * NS
CH = 128
NCHUNK = M // CH
ITERS = (NCHUNK + NW - 1) // NW


def _sc_gather(nidx, eidx, emb, hu):
    mesh = plsc.VectorSubcoreMesh(core_axis_name="c", subcore_axis_name="s")

    @functools.partial(
        pl.kernel,
        mesh=mesh,
        out_type=(
            jax.ShapeDtypeStruct((M, 128), F32),
            jax.ShapeDtypeStruct((M, 256), F32),
        ),
        scratch_types=[
            pltpu.VMEM((CH,), jnp.int32),
            pltpu.VMEM((CH,), jnp.int32),
            pltpu.VMEM((CH, 128), F32),
            pltpu.VMEM((CH, 256), F32),
            pltpu.SemaphoreType.DMA,
            pltpu.SemaphoreType.DMA,
        ],
    )
    def gather_k(nidx_hbm, eidx_hbm, emb_hbm, hu_hbm,
                 ae_out, hu_out,
                 nv, ev, aev, huv, s0, s1):
        wid = lax.axis_index("s") * NC + lax.axis_index("c")

        def body(k, carry):
            c = wid + k * NW

            @pl.when(c < NCHUNK)
            def _():
                off = c * CH
                pltpu.sync_copy(nidx_hbm.at[pl.ds(off, CH)], nv)
                pltpu.sync_copy(eidx_hbm.at[pl.ds(off, CH)], ev)
                ce = pltpu.async_copy(emb_hbm.at[ev], aev, s0)
                cn = pltpu.async_copy(hu_hbm.at[nv], huv, s1)
                ce.wait()
                cn.wait()
                pltpu.sync_copy(aev, ae_out.at[pl.ds(off, CH)])
                pltpu.sync_copy(huv, hu_out.at[pl.ds(off, CH)])

            return carry

        lax.fori_loop(0, ITERS, body, 0)

    return gather_k(nidx, eidx, emb, hu)


MB = 2000
NBLK = M // MB


def _mlp_body(ae_ref, hu_ref,
              w1a_ref, w1b_ref, w1c_ref, b1_ref,
              w2_ref, b2_ref, w3_ref, b3_ref,
              wv_ref, bv_ref,
              av_out, val_out, acc_ref):
    i = pl.program_id(0)

    @pl.when(i == 0)
    def _():
        acc_ref[...] = jnp.zeros_like(acc_ref)

    ae = ae_ref[:, 0:64]
    hg = hu_ref[:, 0:64]
    ug = hu_ref[:, 64:192]
    re = jnp.maximum(ae, 0.0)
    ah = ae * hg
    n2 = jnp.sum(ah * ah, axis=1, keepdims=True)
    ah = ah / jnp.maximum(jnp.sqrt(n2), 1e-12)

    acc_ref[...] += jnp.concatenate(
        [re.sum(axis=0, keepdims=True),
         ah.sum(axis=0, keepdims=True),
         ug.sum(axis=0, keepdims=True)], axis=1)

    x = (jnp.dot(re, w1a_ref[...], preferred_element_type=F32)
         + jnp.dot(ah, w1b_ref[...], preferred_element_type=F32)
         + jnp.dot(ug, w1c_ref[...], preferred_element_type=F32)
         + b1_ref[...])
    x = jnp.maximum(x, 0.0)
    x = jnp.maximum(
        jnp.dot(x, w2_ref[...], preferred_element_type=F32) + b2_ref[...], 0.0)
    av_out[...] = (jnp.dot(x, w3_ref[...], preferred_element_type=F32)
                   + b3_ref[...])

    @pl.when(i == NBLK - 1)
    def _():
        mean = acc_ref[...] * (1.0 / M)
        val_out[...] = (
            jnp.dot(mean, wv_ref[...], preferred_element_type=F32)
            + bv_ref[...])


def _mlp(ae, hu, w1a, w1b, w1c, b1_row, w2t, b2_row, w3t, b3_11, wvt, bv_11):
    return pl.pallas_call(
        _mlp_body,
        grid=(NBLK,),
        in_specs=[
            pl.BlockSpec((MB, 128), lambda i: (i, 0)),
            pl.BlockSpec((MB, 256), lambda i: (i, 0)),
            pl.BlockSpec((64, 64), lambda i: (0, 0)),
            pl.BlockSpec((64, 64), lambda i: (0, 0)),
            pl.BlockSpec((128, 64), lambda i: (0, 0)),
            pl.BlockSpec((1, 64), lambda i: (0, 0)),
            pl.BlockSpec((64, 64), lambda i: (0, 0)),
            pl.BlockSpec((1, 64), lambda i: (0, 0)),
            pl.BlockSpec((64, 1), lambda i: (0, 0)),
            pl.BlockSpec((1, 1), lambda i: (0, 0)),
            pl.BlockSpec((256, 1), lambda i: (0, 0)),
            pl.BlockSpec((1, 1), lambda i: (0, 0)),
        ],
        out_specs=[
            pl.BlockSpec((MB, 1), lambda i: (i, 0)),
            pl.BlockSpec((1, 1), lambda i: (0, 0)),
        ],
        out_shape=[
            jax.ShapeDtypeStruct((M, 1), F32),
            jax.ShapeDtypeStruct((1, 1), F32),
        ],
        scratch_shapes=[pltpu.VMEM((1, 256), F32)],
    )(ae, hu, w1a, w1b, w1c, b1_row, w2t, b2_row, w3t, b3_11, wvt, bv_11)


def kernel(h, cand_edge_feat, u_1, c_adj, j,
           W_phi, b_phi, W1, b1, W2, b2, W3, b3, Wv, bv):
    nidx = c_adj[0]
    eidx = c_adj[1] - j

    w_phi_t_pad = jnp.pad(W_phi.T, ((0, 0), (0, 64)))
    b_phi_row_pad = jnp.pad(b_phi, (0, 64)).reshape(1, 128)
    emb = _edge_embed(cand_edge_feat, w_phi_t_pad, b_phi_row_pad)

    hu = jnp.concatenate([h, u_1, jnp.zeros((N, 64), F32)], axis=1)

    ae, hug = _sc_gather(nidx, eidx, emb, hu)

    w1t = W1.T
    a_values, value = _mlp(
        ae, hug,
        w1t[0:64], w1t[64:128], w1t[128:256], b1.reshape(1, 64),
        W2.T, b2.reshape(1, 64), W3.T, b3.reshape(1, 1),
        Wv.T, bv.reshape(1, 1))

    return (value, a_values.reshape(1, M))


# speedup vs baseline: 3.1795x; 1.4466x over previous
import functools

import jax
import jax.numpy as jnp
from jax import lax
from jax.experimental import pallas as pl
from jax.experimental.pallas import tpu as pltpu
from jax.experimental.pallas import tpu_sc as plsc

F32 = jnp.float32
BF16 = jnp.bfloat16

N = 10000
E = 160000
M = 160000
FD = 128


NC, NS = 2, 16
NW = NC * NS
CH = 128
NCHUNK = M // CH
ITERS = (NCHUNK + NW - 1) // NW


def _sc_gather(nidx, eidx, feat, hu):
    mesh = plsc.VectorSubcoreMesh(core_axis_name="c", subcore_axis_name="s")

    @functools.partial(
        pl.kernel,
        mesh=mesh,
        out_type=(
            jax.ShapeDtypeStruct((M, 128), F32),
            jax.ShapeDtypeStruct((M, 128), jnp.int32),
        ),
        scratch_types=[
            pltpu.VMEM((CH,), jnp.int32),
            pltpu.VMEM((CH,), jnp.int32),
            pltpu.VMEM((CH, 128), F32),
            pltpu.VMEM((CH, 128), jnp.int32),
            pltpu.SemaphoreType.DMA,
            pltpu.SemaphoreType.DMA,
        ],
    )
    def gather_k(nidx_hbm, eidx_hbm, feat_hbm, hu_hbm,
                 fe_out, hu_out,
                 nv, ev, fev, huv, s0, s1):
        wid = lax.axis_index("s") * NC + lax.axis_index("c")

        def body(k, carry):
            c = wid + k * NW

            @pl.when(c < NCHUNK)
            def _():
                off = c * CH
                pltpu.sync_copy(nidx_hbm.at[pl.ds(off, CH)], nv)
                pltpu.sync_copy(eidx_hbm.at[pl.ds(off, CH)], ev)
                ce = pltpu.async_copy(feat_hbm.at[ev], fev, s0)
                cn = pltpu.async_copy(hu_hbm.at[nv], huv, s1)
                ce.wait()
                cn.wait()
                pltpu.sync_copy(fev, fe_out.at[pl.ds(off, CH)])
                pltpu.sync_copy(huv, hu_out.at[pl.ds(off, CH)])

            return carry

        lax.fori_loop(0, ITERS, body, 0)

    return gather_k(nidx, eidx, feat, hu)


MB = 2000
NBLK = M // MB


def _mlp_body(fe_ref, hu_ref,
              wpt_ref, bphi_ref,
              w1a_ref, w1b_ref, w1c_ref, b1_ref,
              w2_ref, b2_ref, w3_ref, b3_ref,
              wv_ref, bv_ref,
              av_out, val_out, acc_ref):
    i = pl.program_id(0)

    @pl.when(i == 0)
    def _():
        acc_ref[...] = jnp.zeros_like(acc_ref)

    ae = (jnp.dot(fe_ref[...], wpt_ref[...], preferred_element_type=F32)
          + bphi_ref[...])
    hu32 = hu_ref[...]
    lo = jax.lax.bitcast_convert_type(hu32 << 16, F32)
    hi = jax.lax.bitcast_convert_type(
        hu32 & jnp.int32(-65536), F32)
    hg = lo[:, 0:64]
    ug = jnp.concatenate([lo[:, 64:128], hi[:, 0:64]], axis=1)
    re = jnp.maximum(ae, 0.0)
    ah = ae * hg
    n2 = jnp.sum(ah * ah, axis=1, keepdims=True)
    ah = ah / jnp.maximum(jnp.sqrt(n2), 1e-12)

    acc_ref[...] += jnp.concatenate(
        [re.sum(axis=0, keepdims=True),
         ah.sum(axis=0, keepdims=True),
         ug.sum(axis=0, keepdims=True)], axis=1)

    x = (jnp.dot(re, w1a_ref[...], preferred_element_type=F32)
         + jnp.dot(ah, w1b_ref[...], preferred_element_type=F32)
         + jnp.dot(ug, w1c_ref[...], preferred_element_type=F32)
         + b1_ref[...])
    x = jnp.maximum(x, 0.0)
    x = jnp.maximum(
        jnp.dot(x, w2_ref[...], preferred_element_type=F32) + b2_ref[...], 0.0)
    av_out[...] = (jnp.dot(x, w3_ref[...], preferred_element_type=F32)
                   + b3_ref[...])

    @pl.when(i == NBLK - 1)
    def _():
        mean = acc_ref[...] * (1.0 / M)
        val_out[...] = (
            jnp.dot(mean, wv_ref[...], preferred_element_type=F32)
            + bv_ref[...])


def _mlp(fe, hu, wpt, bphi_row, w1a, w1b, w1c, b1_row, w2t, b2_row, w3t,
         b3_11, wvt, bv_11):
    return pl.pallas_call(
        _mlp_body,
        grid=(NBLK,),
        in_specs=[
            pl.BlockSpec((MB, 128), lambda i: (i, 0)),
            pl.BlockSpec((MB, 128), lambda i: (i, 0)),
            pl.BlockSpec((128, 64), lambda i: (0, 0)),
            pl.BlockSpec((1, 64), lambda i: (0, 0)),
            pl.BlockSpec((64, 64), lambda i: (0, 0)),
            pl.BlockSpec((64, 64), lambda i: (0, 0)),
            pl.BlockSpec((128, 64), lambda i: (0, 0)),
            pl.BlockSpec((1, 64), lambda i: (0, 0)),
            pl.BlockSpec((64, 64), lambda i: (0, 0)),
            pl.BlockSpec((1, 64), lambda i: (0, 0)),
            pl.BlockSpec((64, 1), lambda i: (0, 0)),
            pl.BlockSpec((1, 1), lambda i: (0, 0)),
            pl.BlockSpec((256, 1), lambda i: (0, 0)),
            pl.BlockSpec((1, 1), lambda i: (0, 0)),
        ],
        out_specs=[
            pl.BlockSpec((MB, 1), lambda i: (i, 0)),
            pl.BlockSpec((1, 1), lambda i: (0, 0)),
        ],
        out_shape=[
            jax.ShapeDtypeStruct((M, 1), F32),
            jax.ShapeDtypeStruct((1, 1), F32),
        ],
        scratch_shapes=[pltpu.VMEM((1, 256), F32)],
    )(fe, hu, wpt, bphi_row, w1a, w1b, w1c, b1_row, w2t, b2_row, w3t,
      b3_11, wvt, bv_11)


def kernel(h, cand_edge_feat, u_1, c_adj, j,
           W_phi, b_phi, W1, b1, W2, b2, W3, b3, Wv, bv):
    nidx = c_adj[0]
    eidx = c_adj[1] - j

    lo_bits = jax.lax.bitcast_convert_type(
        jnp.concatenate([h, u_1[:, 0:64]], axis=1).astype(BF16),
        jnp.uint16).astype(jnp.uint32)
    hi_bits = jax.lax.bitcast_convert_type(
        jnp.concatenate([u_1[:, 64:128], jnp.zeros((N, 64), F32)], axis=1)
        .astype(BF16), jnp.uint16).astype(jnp.uint32)
    hu = jax.lax.bitcast_convert_type(
        lo_bits | (hi_bits << 16), jnp.int32)

    fe, hug = _sc_gather(nidx, eidx, cand_edge_feat, hu)

    w1t = W1.T
    a_values, value = _mlp(
        fe, hug, W_phi.T, b_phi.reshape(1, 64),
        w1t[0:64], w1t[64:128], w1t[128:256], b1.reshape(1, 64),
        W2.T, b2.reshape(1, 64), W3.T, b3.reshape(1, 1),
        Wv.T, bv.reshape(1, 1))

    return (value, a_values.reshape(1, M))


# speedup vs baseline: 3.6299x; 1.1417x over previous
import functools

import jax
import jax.numpy as jnp
from jax import lax
from jax.experimental import pallas as pl
from jax.experimental.pallas import tpu as pltpu
from jax.experimental.pallas import tpu_sc as plsc

F32 = jnp.float32
BF16 = jnp.bfloat16

N = 10000
E = 160000
M = 160000
FD = 128


NC, NS = 2, 16
NW = NC * NS
CH = 128
NCHUNK = M // CH
ITERS = (NCHUNK + NW - 1) // NW


def _sc_gather(nidx, eidx, feat, hu):
    mesh = plsc.VectorSubcoreMesh(core_axis_name="c", subcore_axis_name="s")

    @functools.partial(
        pl.kernel,
        mesh=mesh,
        out_type=(
            jax.ShapeDtypeStruct((M, 128), F32),
            jax.ShapeDtypeStruct((M, 128), jnp.int32),
        ),
        scratch_types=[
            pltpu.VMEM((CH,), jnp.int32),
            pltpu.VMEM((CH,), jnp.int32),
            pltpu.VMEM((CH, 128), F32),
            pltpu.VMEM((CH, 128), jnp.int32),
            pltpu.VMEM((CH,), jnp.int32),
            pltpu.VMEM((CH,), jnp.int32),
            pltpu.VMEM((CH, 128), F32),
            pltpu.VMEM((CH, 128), jnp.int32),
            pltpu.SemaphoreType.DMA,
            pltpu.SemaphoreType.DMA,
            pltpu.SemaphoreType.DMA,
            pltpu.SemaphoreType.DMA,
            pltpu.SemaphoreType.DMA,
            pltpu.SemaphoreType.DMA,
            pltpu.SemaphoreType.DMA,
            pltpu.SemaphoreType.DMA,
        ],
    )
    def gather_k(nidx_hbm, eidx_hbm, feat_hbm, hu_hbm,
                 fe_out, hu_out,
                 nv0, ev0, fv0, uv0, nv1, ev1, fv1, uv1,
                 sf0, sh0, wf0, wh0, sf1, sh1, wf1, wh1):
        wid = lax.axis_index("s") * NC + lax.axis_index("c")
        bufs = [(nv0, ev0, fv0, uv0, sf0, sh0, wf0, wh0),
                (nv1, ev1, fv1, uv1, sf1, sh1, wf1, wh1)]

        def load_and_fire(k, b):
            nv, ev, fv, uv, sf, sh, _, _ = bufs[b]
            c = wid + k * NW

            @pl.when(c < NCHUNK)
            def _():
                off = c * CH
                pltpu.sync_copy(nidx_hbm.at[pl.ds(off, CH)], nv)
                pltpu.sync_copy(eidx_hbm.at[pl.ds(off, CH)], ev)
                pltpu.async_copy(feat_hbm.at[ev], fv, sf)
                pltpu.async_copy(hu_hbm.at[nv], uv, sh)

        def wait_wb(k, b):
            _, _, fv, uv, _, _, wf, wh = bufs[b]
            c = wid + k * NW

            @pl.when(jnp.logical_and(c >= 0, c < NCHUNK))
            def _():
                off = c * CH
                pltpu.make_async_copy(
                    fv, fe_out.at[pl.ds(off, CH)], wf).wait()
                pltpu.make_async_copy(
                    uv, hu_out.at[pl.ds(off, CH)], wh).wait()

        def drain_and_store(k, b):
            nv, ev, fv, uv, sf, sh, wf, wh = bufs[b]
            c = wid + k * NW

            @pl.when(c < NCHUNK)
            def _():
                off = c * CH
                pltpu.make_async_copy(feat_hbm.at[ev], fv, sf).wait()
                pltpu.make_async_copy(hu_hbm.at[nv], uv, sh).wait()
                pltpu.async_copy(fv, fe_out.at[pl.ds(off, CH)], wf)
                pltpu.async_copy(uv, hu_out.at[pl.ds(off, CH)], wh)

        load_and_fire(0, 0)

        def outer(k2, carry):
            for b in (0, 1):
                k = k2 * 2 + b
                wait_wb(k - 1, 1 - b)
                load_and_fire(k + 1, 1 - b)
                drain_and_store(k, b)
            return carry

        lax.fori_loop(0, (ITERS + 1) // 2, outer, 0)

        wait_wb(ITERS - 1, (ITERS - 1) % 2)

    return gather_k(nidx, eidx, feat, hu)


MB = 2000
NBLK = M // MB


def _mlp_body(fe_ref, hu_ref,
              wpt_ref, bphi_ref,
              w1a_ref, w1b_ref, w1c_ref, b1_ref,
              w2_ref, b2_ref, w3_ref, b3_ref,
              wv_ref, bv_ref,
              av_out, val_out, acc_ref):
    i = pl.program_id(0)

    @pl.when(i == 0)
    def _():
        acc_ref[...] = jnp.zeros_like(acc_ref)

    ae = (jnp.dot(fe_ref[...], wpt_ref[...], preferred_element_type=F32)
          + bphi_ref[...])
    hu32 = hu_ref[...]
    lo = jax.lax.bitcast_convert_type(hu32 << 16, F32)
    hi = jax.lax.bitcast_convert_type(
        hu32 & jnp.int32(-65536), F32)
    hg = lo[:, 0:64]
    ug = jnp.concatenate([lo[:, 64:128], hi[:, 0:64]], axis=1)
    re = jnp.maximum(ae, 0.0)
    ah = ae * hg
    n2 = jnp.sum(ah * ah, axis=1, keepdims=True)
    ah = ah / jnp.maximum(jnp.sqrt(n2), 1e-12)

    acc_ref[...] += jnp.concatenate(
        [re.sum(axis=0, keepdims=True),
         ah.sum(axis=0, keepdims=True),
         ug.sum(axis=0, keepdims=True)], axis=1)

    x = (jnp.dot(re, w1a_ref[...], preferred_element_type=F32)
         + jnp.dot(ah, w1b_ref[...], preferred_element_type=F32)
         + jnp.dot(ug, w1c_ref[...], preferred_element_type=F32)
         + b1_ref[...])
    x = jnp.maximum(x, 0.0)
    x = jnp.maximum(
        jnp.dot(x, w2_ref[...], preferred_element_type=F32) + b2_ref[...], 0.0)
    av_out[...] = (jnp.dot(x, w3_ref[...], preferred_element_type=F32)
                   + b3_ref[...])

    @pl.when(i == NBLK - 1)
    def _():
        mean = acc_ref[...] * (1.0 / M)
        val_out[...] = (
            jnp.dot(mean, wv_ref[...], preferred_element_type=F32)
            + bv_ref[...])


def _mlp(fe, hu, wpt, bphi_row, w1a, w1b, w1c, b1_row, w2t, b2_row, w3t,
         b3_11, wvt, bv_11):
    return pl.pallas_call(
        _mlp_body,
        grid=(NBLK,),
        in_specs=[
            pl.BlockSpec((MB, 128), lambda i: (i, 0)),
            pl.BlockSpec((MB, 128), lambda i: (i, 0)),
            pl.BlockSpec((128, 64), lambda i: (0, 0)),
            pl.BlockSpec((1, 64), lambda i: (0, 0)),
            pl.BlockSpec((64, 64), lambda i: (0, 0)),
            pl.BlockSpec((64, 64), lambda i: (0, 0)),
            pl.BlockSpec((128, 64), lambda i: (0, 0)),
            pl.BlockSpec((1, 64), lambda i: (0, 0)),
            pl.BlockSpec((64, 64), lambda i: (0, 0)),
            pl.BlockSpec((1, 64), lambda i: (0, 0)),
            pl.BlockSpec((64, 1), lambda i: (0, 0)),
            pl.BlockSpec((1, 1), lambda i: (0, 0)),
            pl.BlockSpec((256, 1), lambda i: (0, 0)),
            pl.BlockSpec((1, 1), lambda i: (0, 0)),
        ],
        out_specs=[
            pl.BlockSpec((MB, 1), lambda i: (i, 0)),
            pl.BlockSpec((1, 1), lambda i: (0, 0)),
        ],
        out_shape=[
            jax.ShapeDtypeStruct((M, 1), F32),
            jax.ShapeDtypeStruct((1, 1), F32),
        ],
        scratch_shapes=[pltpu.VMEM((1, 256), F32)],
    )(fe, hu, wpt, bphi_row, w1a, w1b, w1c, b1_row, w2t, b2_row, w3t,
      b3_11, wvt, bv_11)


def kernel(h, cand_edge_feat, u_1, c_adj, j,
           W_phi, b_phi, W1, b1, W2, b2, W3, b3, Wv, bv):
    nidx = c_adj[0]
    eidx = c_adj[1] - j

    lo_bits = jax.lax.bitcast_convert_type(
        jnp.concatenate([h, u_1[:, 0:64]], axis=1).astype(BF16),
        jnp.uint16).astype(jnp.uint32)
    hi_bits = jax.lax.bitcast_convert_type(
        jnp.concatenate([u_1[:, 64:128], jnp.zeros((N, 64), F32)], axis=1)
        .astype(BF16), jnp.uint16).astype(jnp.uint32)
    hu = jax.lax.bitcast_convert_type(
        lo_bits | (hi_bits << 16), jnp.int32)

    fe, hug = _sc_gather(nidx, eidx, cand_edge_feat, hu)

    w1t = W1.T
    a_values, value = _mlp(
        fe, hug, W_phi.T, b_phi.reshape(1, 64),
        w1t[0:64], w1t[64:128], w1t[128:256], b1.reshape(1, 64),
        W2.T, b2.reshape(1, 64), W3.T, b3.reshape(1, 1),
        Wv.T, bv.reshape(1, 1))

    return (value, a_values.reshape(1, M))
